# baseline (speedup 1.0000x reference)
import functools

import jax
import jax.numpy as jnp
from jax import lax
from jax.experimental import pallas as pl
from jax.experimental.pallas import tpu as pltpu
from jax.experimental.pallas import tpu_sc as plsc

VOCAB = 100000
N_PREFIX = 200
EMBED_DIM = 64
BATCH = 4096
HIST = 200

NC = 2
NS = 16
NW = NC * NS

ROWS_PER_W = BATCH // NW
GA, GB = 128, HIST - 128
NBUF = 4
NITER = ROWS_PER_W // 2


def _slot(j, b):
    r = 2 * j + (b // 2)
    h0 = 0 if b % 2 == 0 else GA
    n = GA if b % 2 == 0 else GB
    return r, h0, n


def _sc_gather(table, idx):
    mesh = plsc.VectorSubcoreMesh(
        core_axis_name="c", subcore_axis_name="s", num_cores=NC, num_subcores=NS
    )

    @functools.partial(
        pl.kernel,
        out_type=jax.ShapeDtypeStruct((BATCH, HIST, EMBED_DIM), jnp.float32),
        mesh=mesh,
        compiler_params=pltpu.CompilerParams(use_tc_tiling_on_sc=False),
        scratch_types=[
            pltpu.VMEM((ROWS_PER_W, HIST), jnp.int32),
            pltpu.VMEM((NBUF, GA, EMBED_DIM), jnp.float32),
            pltpu.SemaphoreType.DMA,
            pltpu.SemaphoreType.DMA,
            pltpu.SemaphoreType.DMA,
            pltpu.SemaphoreType.DMA,
            pltpu.SemaphoreType.DMA,
            pltpu.SemaphoreType.DMA,
            pltpu.SemaphoreType.DMA,
            pltpu.SemaphoreType.DMA,
        ],
    )
    def body(table_hbm, idx_hbm, out_hbm, idx_v, rows, *sems):
        gsems = sems[:NBUF]
        ssems = sems[NBUF:]
        wid = lax.axis_index("s") * NC + lax.axis_index("c")
        rbase = wid * ROWS_PER_W

        pltpu.sync_copy(idx_hbm.at[pl.ds(rbase, ROWS_PER_W)], idx_v)

        def start_gather(j, b):
            r, h0, n = _slot(j, b)
            pltpu.async_copy(
                table_hbm.at[idx_v.at[r, pl.ds(h0, n)]],
                rows.at[b, pl.ds(0, n)],
                gsems[b],
            )

        def wait_gather(b):
            _, _, n = _slot(0, b)
            pltpu.make_async_copy(
                table_hbm.at[idx_v.at[0, pl.ds(0, n)]],
                rows.at[b, pl.ds(0, n)],
                gsems[b],
            ).wait()

        def start_scatter(j, b):
            r, h0, n = _slot(j, b)
            pltpu.async_copy(
                rows.at[b, pl.ds(0, n)],
                out_hbm.at[rbase + r, pl.ds(h0, n)],
                ssems[b],
            )

        def wait_scatter(b):
            _, h0, n = _slot(0, b)
            pltpu.make_async_copy(
                rows.at[b, pl.ds(0, n)],
                out_hbm.at[rbase, pl.ds(h0, n)],
                ssems[b],
            ).wait()

        start_gather(0, 0)
        start_gather(0, 1)

        def loop(j, carry):
            for b in range(NBUF):
                wait_gather(b)
                start_scatter(j, b)
                b2 = (b + 2) % NBUF
                if b < 2:
                    @pl.when(j > 0)
                    def _():
                        wait_scatter(b2)

                    start_gather(j, b2)
                else:
                    @pl.when(j < NITER - 1)
                    def _():
                        wait_scatter(b2)
                        start_gather(j + 1, b2)

            return carry

        lax.fori_loop(0, NITER, loop, 0)

        for b in range(NBUF):
            wait_scatter(b)

    return body(table, idx)


@jax.jit
def kernel(input, W_embed, W_new):
    table = jnp.concatenate([W_embed, W_new], axis=0)
    idx = input.astype(jnp.int32)
    return _sc_gather(table, idx)


# speedup vs baseline: 1.4090x; 1.4090x over previous
import functools

import jax
import jax.numpy as jnp
from jax import lax
from jax.experimental import pallas as pl
from jax.experimental.pallas import tpu as pltpu
from jax.experimental.pallas import tpu_sc as plsc

VOCAB = 100000
N_PREFIX = 200
EMBED_DIM = 64
BATCH = 4096
HIST = 200
PAD_DIM = 128
HIST_PAD = 256

NC = 2
NS = 16
NW = NC * NS

ROWS_PER_W = BATCH // NW
GA, GB = 128, HIST - 128
NBUF = 4
NITER = ROWS_PER_W // 2


def _slot(j, b):
    r = 2 * j + (b // 2)
    h0 = 0 if b % 2 == 0 else GA
    n = GA if b % 2 == 0 else GB
    return r, h0, n


def _sc_gather(table_pad, idx_pad):
    mesh = plsc.VectorSubcoreMesh(
        core_axis_name="c", subcore_axis_name="s", num_cores=NC, num_subcores=NS
    )

    @functools.partial(
        pl.kernel,
        out_type=jax.ShapeDtypeStruct((BATCH, HIST, PAD_DIM), jnp.float32),
        mesh=mesh,
        compiler_params=pltpu.CompilerParams(use_tc_tiling_on_sc=True),
        scratch_types=[
            pltpu.VMEM((ROWS_PER_W, GA), jnp.int32),
            pltpu.VMEM((ROWS_PER_W, GA), jnp.int32),
            pltpu.VMEM((NBUF, GA, PAD_DIM), jnp.float32),
            pltpu.SemaphoreType.DMA,
            pltpu.SemaphoreType.DMA,
            pltpu.SemaphoreType.DMA,
            pltpu.SemaphoreType.DMA,
            pltpu.SemaphoreType.DMA,
            pltpu.SemaphoreType.DMA,
            pltpu.SemaphoreType.DMA,
            pltpu.SemaphoreType.DMA,
        ],
    )
    def body(table_hbm, idx_hbm, out_hbm, idx_a, idx_b, rows, *sems):
        gsems = sems[:NBUF]
        ssems = sems[NBUF:]
        wid = lax.axis_index("s") * NC + lax.axis_index("c")
        rbase = wid * ROWS_PER_W

        pltpu.sync_copy(idx_hbm.at[pl.ds(rbase, ROWS_PER_W), pl.ds(0, GA)], idx_a)
        pltpu.sync_copy(idx_hbm.at[pl.ds(rbase, ROWS_PER_W), pl.ds(GA, GA)], idx_b)

        def idx_vec(j, b):
            r, h0, n = _slot(j, b)
            src = idx_a if h0 == 0 else idx_b
            return src.at[r, pl.ds(0, n)]

        def start_gather(j, b):
            _, _, n = _slot(j, b)
            pltpu.async_copy(
                table_hbm.at[idx_vec(j, b)],
                rows.at[b, pl.ds(0, n)],
                gsems[b],
            )

        def wait_gather(b):
            _, _, n = _slot(0, b)
            pltpu.make_async_copy(
                table_hbm.at[idx_vec(0, b)],
                rows.at[b, pl.ds(0, n)],
                gsems[b],
            ).wait()

        def start_scatter(j, b):
            r, h0, n = _slot(j, b)
            pltpu.async_copy(
                rows.at[b, pl.ds(0, n)],
                out_hbm.at[rbase + r, pl.ds(h0, n)],
                ssems[b],
            )

        def wait_scatter(b):
            _, h0, n = _slot(0, b)
            pltpu.make_async_copy(
                rows.at[b, pl.ds(0, n)],
                out_hbm.at[rbase, pl.ds(h0, n)],
                ssems[b],
            ).wait()

        start_gather(0, 0)
        start_gather(0, 1)

        def loop(j, carry):
            for b in range(NBUF):
                wait_gather(b)
                start_scatter(j, b)
                b2 = (b + 2) % NBUF
                if b < 2:
                    @pl.when(j > 0)
                    def _():
                        wait_scatter(b2)

                    start_gather(j, b2)
                else:
                    @pl.when(j < NITER - 1)
                    def _():
                        wait_scatter(b2)
                        start_gather(j + 1, b2)

            return carry

        lax.fori_loop(0, NITER, loop, 0)

        for b in range(NBUF):
            wait_scatter(b)

    return body(table_pad, idx_pad)


@jax.jit
def kernel(input, W_embed, W_new):
    table = jnp.concatenate([W_embed, W_new], axis=0)
    table_pad = jnp.pad(table, ((0, 0), (0, PAD_DIM - EMBED_DIM)))
    idx_pad = jnp.pad(input.astype(jnp.int32), ((0, 0), (0, HIST_PAD - HIST)))
    out = _sc_gather(table_pad, idx_pad)
    return out[:, :, :EMBED_DIM]
